# Initial kernel scaffold; baseline (speedup 1.0000x reference)
import jax
import jax.numpy as jnp
from jax.experimental import pallas as pl


def kernel(edge_vec, center_index, neighbor_species_index):
    raise NotImplementedError("write your pallas kernel here")


# speedup vs baseline: 2.2324x; 2.2324x over previous
import math

import jax
import jax.numpy as jnp
from jax import lax
from jax.experimental import pallas as pl
from jax.experimental.pallas import tpu as pltpu
from jax.experimental.pallas import tpu_sc as plsc

R_CUT = 5.0
N_MAX_L = [8, 7, 6, 5, 4]
OFFS = [0, 8, 15, 21, 26]
N_SPECIES = 4
N_CENTERS = 50000
NSEG = N_CENTERS * N_SPECIES
N_EDGES = 1600000

NC = 2
NS = 16
LANES = 16

CHUNK = 128
BLOCK = 2048
BLOCKS_PER_TEC = 49
E_PAD = NS * BLOCKS_PER_TEC * BLOCK
ROWS_PER_TEC = NSEG // NS
ZROWS = 500

_COLMAP = [(l, n) for l in range(5) for n in range(1, N_MAX_L[l] + 1)]
_GROUPS = [[_COLMAP[8 * g + t] if 8 * g + t < 30 else None for t in range(8)]
           for g in range(4)]

_A = math.pi / 2.0
_SIN_C = [(-1.0) ** k * _A ** (2 * k + 1) / math.factorial(2 * k + 1)
          for k in range(6)]
_COS_C = [(-1.0) ** k * _A ** (2 * k) / math.factorial(2 * k)
          for k in range(7)]


def _poly(u, coeffs):
    acc = jnp.full((LANES,), coeffs[-1], jnp.float32)
    for c in reversed(coeffs[:-1]):
        acc = acc * u + c
    return acc


def _features(ex, ey, ez, valid):
    r2 = ex * ex + ey * ey + ez * ez
    r2 = jnp.maximum(r2, 1e-24)
    i = plsc.bitcast(r2, jnp.int32)
    i = jnp.full((LANES,), 0x5F3759DF, jnp.int32) - lax.shift_right_logical(
        i, jnp.full((LANES,), 1, jnp.int32))
    y = plsc.bitcast(i, jnp.float32)
    for _ in range(3):
        y = y * (1.5 - 0.5 * r2 * y * y)
    r = r2 * y
    q = 1.0 / (r + 1e-12)
    q = jnp.where(valid, q, 0.0)
    x = jnp.minimum(r * (1.0 / R_CUT), 1.0)
    u = x * x
    sh = x * _poly(u, _SIN_C)
    ch = _poly(u, _COS_C)
    s1 = 2.0 * sh * ch
    c1 = 1.0 - 2.0 * sh * sh
    two_c1 = c1 + c1
    raw = [s1, two_c1 * s1]
    for _ in range(3, 9):
        raw.append(two_c1 * raw[-1] - raw[-2])
    A = [rr * q for rr in raw[:8]]
    cp = [ch]
    for _ in range(4):
        cp.append(cp[-1] * ch)
    return A, cp


def _col_val(A, cp, spec):
    if spec is None:
        return jnp.zeros((LANES,), jnp.float32)
    l, n = spec
    return A[n - 1] * cp[l]


import functools


@functools.lru_cache(maxsize=1)
def _make_sc_call():
    mesh = plsc.VectorSubcoreMesh(core_axis_name="c", subcore_axis_name="s",
                                  num_cores=NC, num_subcores=NS)
    cparams = pltpu.CompilerParams(needs_layout_passes=False,
                                   use_tc_tiling_on_sc=False)

    @pl.kernel(
        out_type=jax.ShapeDtypeStruct((4, NSEG, 8), jnp.float32),
        mesh=mesh,
        scratch_types=[
            pltpu.VMEM((BLOCK, 3), jnp.float32),
            pltpu.VMEM((BLOCK,), jnp.int32),
            pltpu.VMEM((BLOCK,), jnp.int32),
            pltpu.VMEM((16, CHUNK), jnp.int32),
            pltpu.VMEM((CHUNK, 8), jnp.float32),
            pltpu.VMEM((ZROWS, 8), jnp.float32),
            pltpu.VMEM_SHARED((NSEG, 8), jnp.float32),
        ],
        compiler_params=cparams,
    )
    def sc_call(ev_hbm, ci_hbm, si_hbm, out_hbm,
                ev_v, ci_v, si_v, didx_v, rows_v, zbuf_v, table_sh):
        c = lax.axis_index("c")
        w = lax.axis_index("s")
        cvec = jnp.full((LANES,), 1.0, jnp.float32) * lax.convert_element_type(
            c, jnp.float32)
        lane = lax.iota(jnp.int32, LANES)
        zeros16 = jnp.zeros((LANES,), jnp.float32)

        rhalf = lax.shift_right_logical(lane, jnp.full((LANES,), 3, jnp.int32))
        c8 = lane & 7

        @pl.loop(0, ZROWS // 2)
        def _(i):
            plsc.store_scatter(zbuf_v, [i * 2 + rhalf, c8], zeros16)

        for p in range(2):
            @pl.loop(0, ROWS_PER_TEC // ZROWS)
            def _(i):
                pltpu.sync_copy(
                    zbuf_v,
                    table_sh.at[pl.ds(w * ROWS_PER_TEC + i * ZROWS, ZROWS)])

            plsc.subcore_barrier()

            @pl.loop(0, BLOCKS_PER_TEC)
            def _(b):
                base = (w * BLOCKS_PER_TEC + b) * BLOCK
                pltpu.sync_copy(ev_hbm.at[pl.ds(base, BLOCK)], ev_v)
                pltpu.sync_copy(ci_hbm.at[pl.ds(base, BLOCK)], ci_v)
                pltpu.sync_copy(si_hbm.at[pl.ds(base, BLOCK)], si_v)

                @pl.loop(0, 16)
                def _(k):
                    for j in range(8):
                        o = k * CHUNK + j * LANES
                        rows = o + lane
                        cidx = ci_v[pl.ds(o, LANES)]
                        sidx = si_v[pl.ds(o, LANES)]
                        didx_v[k, pl.ds(j * LANES, LANES)] = (
                            cidx * N_SPECIES + sidx)
                        ex = plsc.load_gather(
                            ev_v, [rows, jnp.zeros((LANES,), jnp.int32)])
                        ey = plsc.load_gather(
                            ev_v, [rows, jnp.full((LANES,), 1, jnp.int32)])
                        ez = plsc.load_gather(
                            ev_v, [rows, jnp.full((LANES,), 2, jnp.int32)])
                        valid = (base + o + lane) < N_EDGES
                        A, cp = _features(ex, ey, ez, valid)
                        ridx = j * LANES + lane
                        for t in range(8):
                            va = _col_val(A, cp, _GROUPS[p][t])
                            vb = _col_val(A, cp, _GROUPS[2 + p][t])
                            val = va + cvec * (vb - va)
                            plsc.store_scatter(
                                rows_v, [ridx, jnp.full((LANES,), t, jnp.int32)],
                                val)
                    pltpu.sync_copy(rows_v, table_sh.at[didx_v.at[k]], add=True)

            plsc.subcore_barrier()

            g = 2 * c + p
            pltpu.sync_copy(
                table_sh.at[pl.ds(w * ROWS_PER_TEC, ROWS_PER_TEC)],
                out_hbm.at[g, pl.ds(w * ROWS_PER_TEC, ROWS_PER_TEC)])

            plsc.subcore_barrier()

    return sc_call


def kernel(edge_vec, center_index, neighbor_species_index):
    pad = E_PAD - N_EDGES
    ev = jnp.pad(edge_vec, ((0, pad), (0, 0)))
    ci = jnp.pad(center_index, ((0, pad),))
    si = jnp.pad(neighbor_species_index, ((0, pad),))
    tbl = _make_sc_call()(ev, ci, si)
    d = tbl.transpose(1, 0, 2).reshape(NSEG, 32)[:, :30]
    d = d.reshape(N_CENTERS, N_SPECIES, 30)
    return jnp.concatenate(
        [d[:, :, OFFS[l]:OFFS[l] + N_MAX_L[l]].reshape(
            N_CENTERS, N_SPECIES * N_MAX_L[l]) for l in range(5)],
        axis=1)


# speedup vs baseline: 8.2898x; 3.7133x over previous
import functools
import math

import jax
import jax.numpy as jnp
from jax import lax
from jax.experimental import pallas as pl
from jax.experimental.pallas import tpu as pltpu
from jax.experimental.pallas import tpu_sc as plsc

R_CUT = 5.0
N_MAX_L = [8, 7, 6, 5, 4]
OFFS = [0, 8, 15, 21, 26]
N_SPECIES = 4
N_CENTERS = 50000
NSEG = N_CENTERS * N_SPECIES
N_EDGES = 1600000

NC = 2
NS = 16
LANES = 16

CHUNK = 128
BLOCK = 2048
NB = 50
E_PAD = NS * NB * BLOCK
ROWS_PER_TEC = NSEG // NS
ZROWS = 500
NRING = 2

_COLMAP = [(l, n) for l in range(5) for n in range(1, N_MAX_L[l] + 1)]
_GROUPS = [[_COLMAP[8 * g + t] if 8 * g + t < 30 else None for t in range(8)]
           for g in range(4)]

_A = math.pi / 2.0
_SIN_C = [(-1.0) ** k * _A ** (2 * k + 1) / math.factorial(2 * k + 1)
          for k in range(6)]
_COS_C = [(-1.0) ** k * _A ** (2 * k) / math.factorial(2 * k)
          for k in range(7)]


def _poly(u, coeffs):
    acc = jnp.full((LANES,), coeffs[-1], jnp.float32)
    for c in reversed(coeffs[:-1]):
        acc = acc * u + c
    return acc


def _features(ex, ey, ez, valid):
    r2 = ex * ex + ey * ey + ez * ez
    r2 = jnp.maximum(r2, 1e-24)
    i = plsc.bitcast(r2, jnp.int32)
    i = jnp.full((LANES,), 0x5F3759DF, jnp.int32) - lax.shift_right_logical(
        i, jnp.full((LANES,), 1, jnp.int32))
    y = plsc.bitcast(i, jnp.float32)
    for _ in range(3):
        y = y * (1.5 - 0.5 * r2 * y * y)
    r = r2 * y
    q = 1.0 / (r + 1e-12)
    q = jnp.where(valid, q, 0.0)
    x = jnp.minimum(r * (1.0 / R_CUT), 1.0)
    u = x * x
    sh = x * _poly(u, _SIN_C)
    ch = _poly(u, _COS_C)
    s1 = 2.0 * sh * ch
    c1 = 1.0 - 2.0 * sh * sh
    two_c1 = c1 + c1
    raw = [s1, two_c1 * s1]
    for _ in range(3, 9):
        raw.append(two_c1 * raw[-1] - raw[-2])
    A = [rr * q for rr in raw[:8]]
    cp = [ch]
    for _ in range(4):
        cp.append(cp[-1] * ch)
    return A, cp


def _col_val(A, cp, spec):
    if spec is None:
        return jnp.zeros((LANES,), jnp.float32)
    l, n = spec
    return A[n - 1] * cp[l]


@functools.lru_cache(maxsize=1)
def _make_sc_call():
    mesh = plsc.VectorSubcoreMesh(core_axis_name="c", subcore_axis_name="s",
                                  num_cores=NC, num_subcores=NS)
    cparams = pltpu.CompilerParams(needs_layout_passes=False,
                                   use_tc_tiling_on_sc=False)

    @pl.kernel(
        out_type=jax.ShapeDtypeStruct((4, NSEG, 8), jnp.float32),
        mesh=mesh,
        scratch_types=[
            pltpu.VMEM((2, BLOCK), jnp.float32),
            pltpu.VMEM((2, BLOCK), jnp.float32),
            pltpu.VMEM((2, BLOCK), jnp.float32),
            pltpu.VMEM((2, BLOCK), jnp.int32),
            pltpu.VMEM((2, BLOCK), jnp.int32),
            pltpu.VMEM((16, CHUNK), jnp.int32),
            pltpu.VMEM((NRING, CHUNK, 8), jnp.float32),
            pltpu.VMEM((ZROWS, 8), jnp.float32),
            pltpu.VMEM_SHARED((NSEG, 8), jnp.float32),
            pltpu.SemaphoreType.DMA((2,)),
            pltpu.SemaphoreType.DMA((NRING,)),
        ],
        compiler_params=cparams,
    )
    def sc_call(evx_hbm, evy_hbm, evz_hbm, ci_hbm, si_hbm, out_hbm,
                evx_v, evy_v, evz_v, ci_v, si_v, didx_v, rows_v, zbuf_v,
                table_sh, in_sem, sc_sem):
        c = lax.axis_index("c")
        w = lax.axis_index("s")
        cvec = jnp.full((LANES,), 1.0, jnp.float32) * lax.convert_element_type(
            c, jnp.float32)
        lane = lax.iota(jnp.int32, LANES)
        zeros16 = jnp.zeros((LANES,), jnp.float32)
        tfull = [jnp.full((LANES,), t, jnp.int32) for t in range(8)]

        def in_copies(b, d):
            base = (w * NB + b) * BLOCK
            sl = pl.ds(base, BLOCK)
            return [
                pltpu.make_async_copy(evx_hbm.at[sl], evx_v.at[d], in_sem.at[d]),
                pltpu.make_async_copy(evy_hbm.at[sl], evy_v.at[d], in_sem.at[d]),
                pltpu.make_async_copy(evz_hbm.at[sl], evz_v.at[d], in_sem.at[d]),
                pltpu.make_async_copy(ci_hbm.at[sl], ci_v.at[d], in_sem.at[d]),
                pltpu.make_async_copy(si_hbm.at[sl], si_v.at[d], in_sem.at[d]),
            ]

        rhalf = lax.shift_right_logical(lane, jnp.full((LANES,), 3, jnp.int32))
        c8 = lane & 7

        @pl.loop(0, ZROWS // 2)
        def _(i):
            plsc.store_scatter(zbuf_v, [i * 2 + rhalf, c8], zeros16)

        for p in range(2):
            @pl.loop(0, ROWS_PER_TEC // ZROWS)
            def _(i):
                pltpu.sync_copy(
                    zbuf_v,
                    table_sh.at[pl.ds(w * ROWS_PER_TEC + i * ZROWS, ZROWS)])

            plsc.subcore_barrier()

            for d in range(2):
                for cp_ in in_copies(d, d):
                    cp_.start()

            @pl.loop(0, NB, step=2)
            def _(b0):
                for d in range(2):
                    b = b0 + d
                    for cp_ in in_copies(b, d):
                        cp_.wait()

                    @pl.loop(0, 16, step=NRING)
                    def _(k0):
                        for qq in range(NRING):
                            k = k0 + qq

                            @pl.when(k0 > 0)
                            def _():
                                pltpu.make_async_copy(
                                    rows_v.at[qq],
                                    table_sh.at[didx_v.at[k - NRING]],
                                    sc_sem.at[qq]).wait()

                            for j in range(8):
                                o = k * CHUNK + j * LANES
                                cidx = ci_v[d, pl.ds(o, LANES)]
                                sidx = si_v[d, pl.ds(o, LANES)]
                                didx_v[k, pl.ds(j * LANES, LANES)] = (
                                    cidx * N_SPECIES + sidx)
                                ex = evx_v[d, pl.ds(o, LANES)]
                                ey = evy_v[d, pl.ds(o, LANES)]
                                ez = evz_v[d, pl.ds(o, LANES)]
                                valid = ((w * NB + b) * BLOCK + o + lane) < N_EDGES
                                A, cpw = _features(ex, ey, ez, valid)
                                ridx = j * LANES + lane
                                for t in range(8):
                                    va = _col_val(A, cpw, _GROUPS[p][t])
                                    vb = _col_val(A, cpw, _GROUPS[2 + p][t])
                                    val = va + cvec * (vb - va)
                                    plsc.store_scatter(
                                        rows_v.at[qq], [ridx, tfull[t]], val)
                            pltpu.async_copy(
                                rows_v.at[qq], table_sh.at[didx_v.at[k]],
                                sc_sem.at[qq], add=True)

                    for qq in range(NRING):
                        pltpu.make_async_copy(
                            rows_v.at[qq],
                            table_sh.at[didx_v.at[16 - NRING + qq]],
                            sc_sem.at[qq]).wait()

                    @pl.when(b + 2 < NB)
                    def _():
                        for cp_ in in_copies(b + 2, d):
                            cp_.start()

            plsc.subcore_barrier()

            g = 2 * c + p
            pltpu.sync_copy(
                table_sh.at[pl.ds(w * ROWS_PER_TEC, ROWS_PER_TEC)],
                out_hbm.at[g, pl.ds(w * ROWS_PER_TEC, ROWS_PER_TEC)])

            plsc.subcore_barrier()

    return sc_call


def kernel(edge_vec, center_index, neighbor_species_index):
    pad = E_PAD - N_EDGES
    evx = jnp.pad(edge_vec[:, 0], ((0, pad),))
    evy = jnp.pad(edge_vec[:, 1], ((0, pad),))
    evz = jnp.pad(edge_vec[:, 2], ((0, pad),))
    ci = jnp.pad(center_index, ((0, pad),))
    si = jnp.pad(neighbor_species_index, ((0, pad),))
    tbl = _make_sc_call()(evx, evy, evz, ci, si)
    d = tbl.transpose(1, 0, 2).reshape(NSEG, 32)[:, :30]
    d = d.reshape(N_CENTERS, N_SPECIES, 30)
    return jnp.concatenate(
        [d[:, :, OFFS[l]:OFFS[l] + N_MAX_L[l]].reshape(
            N_CENTERS, N_SPECIES * N_MAX_L[l]) for l in range(5)],
        axis=1)
